# Initial kernel scaffold; baseline (speedup 1.0000x reference)
import functools

import jax
import jax.numpy as jnp
from jax import lax
from jax.experimental import pallas as pl
from jax.experimental.pallas import tpu as pltpu
from jax.experimental.pallas import tpu_sc as plsc

N = 10000
NP = 10240
E = 320000
D_IN = 128
DH = 16
D_OUT = 7
NEG = 0.2

NC = 2
NS = 16
NW = NC * NS
EW = E // NW
C = 80
CHUNKS = EW // C
ROWS_PER_TILE = NP // NS


def _prep1_body(x_ref, w1_ref, as1_ref, ad1_ref, ew_ref,
                h_ref, a1s_ref, a1d_ref, lew_ref):
    h = jnp.dot(x_ref[:], w1_ref[:], preferred_element_type=jnp.float32)
    h_ref[:] = h
    a1s_ref[:] = jnp.dot(h, as1_ref[:], preferred_element_type=jnp.float32)
    a1d_ref[:] = jnp.dot(h, ad1_ref[:], preferred_element_type=jnp.float32)
    lew_ref[:] = jnp.log2(ew_ref[:])


def _comb1_body(op_ref, sp_ref, a1s_ref, a1d_ref, h1_ref, b1_ref,
                w2_ref, as2_ref, ad2_ref,
                h2_ref, a2s_ref, a2d_ref):
    a = a1s_ref[:] + a1d_ref[:]
    selfw = jnp.exp(jnp.where(a >= 0, a, a * NEG))
    sp = sp_ref[:]
    s = sp[:NP] + sp[NP:] + selfw
    op = op_ref[:]
    num = op[:NP] + op[NP:] + selfw * h1_ref[:]
    z = jnp.maximum(num / s + b1_ref[:], 0.0)
    h2 = jnp.dot(z, w2_ref[:], preferred_element_type=jnp.float32)
    h2_ref[:] = h2
    a2s_ref[:] = jnp.dot(h2, as2_ref[:], preferred_element_type=jnp.float32)
    a2d_ref[:] = jnp.dot(h2, ad2_ref[:], preferred_element_type=jnp.float32)


def _comb2_body(op_ref, sp_ref, a2s_ref, a2d_ref, h2_ref, b2_ref, out_ref):
    a = a2s_ref[:] + a2d_ref[:]
    selfw = jnp.exp(jnp.where(a >= 0, a, a * NEG))
    sp = sp_ref[:]
    s = sp[:NP] + sp[NP:] + selfw
    op = op_ref[:]
    num = op[:NP] + op[NP:] + selfw * h2_ref[:]
    out_ref[:] = num / s + b2_ref[:]


def _edge_body(src_h, dst_h, lew_h, asrc_h, adst_h, h_h, zrows_h, zs_h,
               outp_h, sp_h,
               asrc_v, adst_v, src_v, dst_v, lew_v, w_v, rows_v,
               out_sh, s_sh, sem):
    c = lax.axis_index("c")
    s = lax.axis_index("s")
    wid = s * NC + c

    pltpu.sync_copy(asrc_h, asrc_v)
    pltpu.sync_copy(adst_h, adst_v)

    r0 = s * ROWS_PER_TILE
    pltpu.sync_copy(zrows_h.at[pl.ds(r0, ROWS_PER_TILE)],
                    out_sh.at[pl.ds(r0, ROWS_PER_TILE)])
    pltpu.sync_copy(zs_h.at[pl.ds(r0, ROWS_PER_TILE)],
                    s_sh.at[pl.ds(r0, ROWS_PER_TILE)])
    plsc.subcore_barrier()

    ebase = wid * EW

    def chunk(i, carry):
        base = ebase + i * C
        pltpu.sync_copy(src_h.at[pl.ds(base, C)], src_v)
        pltpu.sync_copy(dst_h.at[pl.ds(base, C)], dst_v)
        pltpu.sync_copy(lew_h.at[pl.ds(base, C)], lew_v)
        gat = pltpu.async_copy(h_h.at[src_v], rows_v, sem)
        for g in range(C // 16):
            sl = pl.ds(g * 16, 16)
            si = src_v[sl]
            di = dst_v[sl]
            a = plsc.load_gather(asrc_v, [si]) + plsc.load_gather(adst_v, [di])
            a = jnp.where(a >= 0, a, a * NEG) + lew_v[sl]
            w_v[sl] = jnp.exp(a)
        gat.wait()
        for e in range(C):
            rows_v[e, :] = rows_v[e, :] * w_v[e]
        pltpu.sync_copy(w_v, s_sh.at[dst_v], add=True)
        pltpu.sync_copy(rows_v, out_sh.at[dst_v], add=True)
        return carry

    lax.fori_loop(0, CHUNKS, chunk, 0)
    plsc.subcore_barrier()

    @pl.when(s == 0)
    def _():
        pltpu.sync_copy(out_sh, outp_h.at[pl.ds(c * NP, NP)])
        pltpu.sync_copy(s_sh, sp_h.at[pl.ds(c * NP, NP)])


_edge_pass = functools.partial(
    pl.kernel,
    out_type=(jax.ShapeDtypeStruct((2 * NP, DH), jnp.float32),
              jax.ShapeDtypeStruct((2 * NP,), jnp.float32)),
    mesh=plsc.VectorSubcoreMesh(core_axis_name="c", subcore_axis_name="s",
                                num_cores=NC, num_subcores=NS),
    scratch_types=[
        pltpu.VMEM((NP,), jnp.float32),
        pltpu.VMEM((NP,), jnp.float32),
        pltpu.VMEM((C,), jnp.int32),
        pltpu.VMEM((C,), jnp.int32),
        pltpu.VMEM((C,), jnp.float32),
        pltpu.VMEM((C,), jnp.float32),
        pltpu.VMEM((C, DH), jnp.float32),
        pltpu.VMEM_SHARED((NP, DH), jnp.float32),
        pltpu.VMEM_SHARED((NP,), jnp.float32),
        pltpu.SemaphoreType.DMA,
    ],
)(_edge_body)


def kernel(x, edge_index, edge_weight, W1, att_src1, att_dst1, b1,
           W2, att_src2, att_dst2, b2):
    f32 = jnp.float32
    src = edge_index[0]
    dst = edge_index[1]
    xp = jnp.pad(x, ((0, NP - N), (0, 0)))
    ew2d = edge_weight.reshape(E // 128, 128)

    h1, a1s, a1d, lew2d = pl.pallas_call(
        _prep1_body,
        out_shape=(jax.ShapeDtypeStruct((NP, DH), f32),
                   jax.ShapeDtypeStruct((NP, 1), f32),
                   jax.ShapeDtypeStruct((NP, 1), f32),
                   jax.ShapeDtypeStruct((E // 128, 128), f32)),
    )(xp, W1, att_src1.reshape(DH, 1), att_dst1.reshape(DH, 1), ew2d)

    lew = lew2d.reshape(E)
    zrows = jnp.zeros((NP, DH), f32)
    zs = jnp.zeros((NP,), f32)

    outp1, sp1 = _edge_pass(src, dst, lew, a1s.reshape(NP), a1d.reshape(NP),
                            h1, zrows, zs)

    W2p = jnp.pad(W2, ((0, 0), (0, DH - D_OUT)))
    as2p = jnp.pad(att_src2, (0, DH - D_OUT)).reshape(DH, 1)
    ad2p = jnp.pad(att_dst2, (0, DH - D_OUT)).reshape(DH, 1)

    h2, a2s, a2d = pl.pallas_call(
        _comb1_body,
        out_shape=(jax.ShapeDtypeStruct((NP, DH), f32),
                   jax.ShapeDtypeStruct((NP, 1), f32),
                   jax.ShapeDtypeStruct((NP, 1), f32)),
    )(outp1, sp1.reshape(2 * NP, 1), a1s, a1d, h1, b1.reshape(1, DH),
      W2p, as2p, ad2p)

    outp2, sp2 = _edge_pass(src, dst, lew, a2s.reshape(NP), a2d.reshape(NP),
                            h2, zrows, zs)

    b2p = jnp.pad(b2, (0, DH - D_OUT))
    out_full = pl.pallas_call(
        _comb2_body,
        out_shape=jax.ShapeDtypeStruct((NP, DH), f32),
    )(outp2, sp2.reshape(2 * NP, 1), a2s, a2d, h2, b2p.reshape(1, DH))

    return out_full[:N, :D_OUT]


# speedup vs baseline: 31.5107x; 31.5107x over previous
import functools

import jax
import jax.numpy as jnp
from jax import lax
from jax.experimental import pallas as pl
from jax.experimental.pallas import tpu as pltpu
from jax.experimental.pallas import tpu_sc as plsc

N = 10000
NP = 10240
E = 320000
D_IN = 128
DH = 16
D_OUT = 7
NEG = 0.2

NC = 2
NS = 16
NW = NC * NS
EW = E // NW
C = 80
CHUNKS = EW // C
ROWS_PER_TILE = NP // NS


def _prep1_body(x_ref, w1_ref, as1_ref, ad1_ref, ew_ref,
                h_ref, a1s_ref, a1d_ref, lew_ref):
    h = jnp.dot(x_ref[:], w1_ref[:], preferred_element_type=jnp.float32)
    h_ref[:] = h
    a1s_ref[:] = jnp.dot(h, as1_ref[:], preferred_element_type=jnp.float32)
    a1d_ref[:] = jnp.dot(h, ad1_ref[:], preferred_element_type=jnp.float32)
    lew_ref[:] = jnp.log2(ew_ref[:])


def _comb1_body(op_ref, sp_ref, a1s_ref, a1d_ref, h1_ref, b1_ref,
                w2_ref, as2_ref, ad2_ref,
                h2_ref, a2s_ref, a2d_ref):
    a = a1s_ref[:] + a1d_ref[:]
    selfw = jnp.exp(jnp.where(a >= 0, a, a * NEG))
    s = (sp_ref[0, :] + sp_ref[1, :]).reshape(NP, 1) + selfw
    op = op_ref[:]
    num = op[:NP] + op[NP:] + selfw * h1_ref[:]
    z = jnp.maximum(num / s + b1_ref[:], 0.0)
    h2 = jnp.dot(z, w2_ref[:], preferred_element_type=jnp.float32)
    h2_ref[:] = h2
    a2s_ref[:] = jnp.dot(h2, as2_ref[:], preferred_element_type=jnp.float32)
    a2d_ref[:] = jnp.dot(h2, ad2_ref[:], preferred_element_type=jnp.float32)


def _comb2_body(op_ref, sp_ref, a2s_ref, a2d_ref, h2_ref, b2_ref, out_ref):
    a = a2s_ref[:] + a2d_ref[:]
    selfw = jnp.exp(jnp.where(a >= 0, a, a * NEG))
    s = (sp_ref[0, :] + sp_ref[1, :]).reshape(NP, 1) + selfw
    op = op_ref[:]
    num = op[:NP] + op[NP:] + selfw * h2_ref[:]
    out_ref[:] = num / s + b2_ref[:]


def _edge_body(src_h, dst_h, lew_h, asrc_h, adst_h, h_h, zrows_h, zs_h,
               outp_h, sp_h,
               asrc_v, adst_v, src_v, dst_v, lew_v, w_v, rows_v,
               out_sh, s_sh, sem):
    c = lax.axis_index("c")
    s = lax.axis_index("s")
    wid = s * NC + c

    pltpu.sync_copy(asrc_h, asrc_v)
    pltpu.sync_copy(adst_h, adst_v)

    r0 = s * ROWS_PER_TILE
    pltpu.sync_copy(zrows_h.at[pl.ds(r0, ROWS_PER_TILE)],
                    out_sh.at[pl.ds(r0, ROWS_PER_TILE)])
    pltpu.sync_copy(zs_h.at[pl.ds(r0, ROWS_PER_TILE)],
                    s_sh.at[pl.ds(r0, ROWS_PER_TILE)])
    plsc.subcore_barrier()

    ebase = wid * EW

    def chunk(i, carry):
        base = ebase + i * C
        pltpu.sync_copy(src_h.at[pl.ds(base, C)], src_v)
        pltpu.sync_copy(dst_h.at[pl.ds(base, C)], dst_v)
        pltpu.sync_copy(lew_h.at[pl.ds(base, C)], lew_v)
        gat = pltpu.async_copy(h_h.at[src_v], rows_v, sem)
        for g in range(C // 16):
            sl = pl.ds(g * 16, 16)
            si = src_v[sl]
            di = dst_v[sl]
            a = plsc.load_gather(asrc_v, [si]) + plsc.load_gather(adst_v, [di])
            a = jnp.where(a >= 0, a, a * NEG) + lew_v[sl]
            w_v[sl] = jnp.exp(a)
        gat.wait()
        for g in range(C // 16):
            wv = w_v[pl.ds(g * 16, 16)]
            for j in range(16):
                e = g * 16 + j
                rows_v[e, :] = rows_v[e, :] * wv[j]
        pltpu.sync_copy(w_v, s_sh.at[dst_v], add=True)
        pltpu.sync_copy(rows_v, out_sh.at[dst_v], add=True)
        return carry

    lax.fori_loop(0, CHUNKS, chunk, 0)
    plsc.subcore_barrier()

    @pl.when(s == 0)
    def _():
        pltpu.sync_copy(out_sh, outp_h.at[pl.ds(c * NP, NP)])
        pltpu.sync_copy(s_sh, sp_h.at[pl.ds(c * NP, NP)])


_edge_pass = functools.partial(
    pl.kernel,
    out_type=(jax.ShapeDtypeStruct((2 * NP, DH), jnp.float32),
              jax.ShapeDtypeStruct((2 * NP,), jnp.float32)),
    mesh=plsc.VectorSubcoreMesh(core_axis_name="c", subcore_axis_name="s",
                                num_cores=NC, num_subcores=NS),
    scratch_types=[
        pltpu.VMEM((NP,), jnp.float32),
        pltpu.VMEM((NP,), jnp.float32),
        pltpu.VMEM((C,), jnp.int32),
        pltpu.VMEM((C,), jnp.int32),
        pltpu.VMEM((C,), jnp.float32),
        pltpu.VMEM((C,), jnp.float32),
        pltpu.VMEM((C, DH), jnp.float32),
        pltpu.VMEM_SHARED((NP, DH), jnp.float32),
        pltpu.VMEM_SHARED((NP,), jnp.float32),
        pltpu.SemaphoreType.DMA,
    ],
    compiler_params=pltpu.CompilerParams(needs_layout_passes=False,
                                         use_tc_tiling_on_sc=False),
)(_edge_body)


def kernel(x, edge_index, edge_weight, W1, att_src1, att_dst1, b1,
           W2, att_src2, att_dst2, b2):
    f32 = jnp.float32
    src = edge_index[0]
    dst = edge_index[1]
    xp = jnp.pad(x, ((0, NP - N), (0, 0)))
    ew2d = edge_weight.reshape(E // 128, 128)

    h1, a1s, a1d, lew2d = pl.pallas_call(
        _prep1_body,
        out_shape=(jax.ShapeDtypeStruct((NP, DH), f32),
                   jax.ShapeDtypeStruct((NP, 1), f32),
                   jax.ShapeDtypeStruct((NP, 1), f32),
                   jax.ShapeDtypeStruct((E // 128, 128), f32)),
    )(xp, W1, att_src1.reshape(DH, 1), att_dst1.reshape(DH, 1), ew2d)

    lew = lew2d.reshape(E)
    zrows = jnp.zeros((NP, DH), f32)
    zs = jnp.zeros((NP,), f32)

    outp1, sp1 = _edge_pass(src, dst, lew, a1s.reshape(NP), a1d.reshape(NP),
                            h1, zrows, zs)

    W2p = jnp.pad(W2, ((0, 0), (0, DH - D_OUT)))
    as2p = jnp.pad(att_src2, (0, DH - D_OUT)).reshape(DH, 1)
    ad2p = jnp.pad(att_dst2, (0, DH - D_OUT)).reshape(DH, 1)

    h2, a2s, a2d = pl.pallas_call(
        _comb1_body,
        out_shape=(jax.ShapeDtypeStruct((NP, DH), f32),
                   jax.ShapeDtypeStruct((NP, 1), f32),
                   jax.ShapeDtypeStruct((NP, 1), f32)),
        compiler_params=pltpu.CompilerParams(vmem_limit_bytes=110 * 2**20),
    )(outp1, sp1.reshape(2, NP), a1s, a1d, h1, b1.reshape(1, DH),
      W2p, as2p, ad2p)

    outp2, sp2 = _edge_pass(src, dst, lew, a2s.reshape(NP), a2d.reshape(NP),
                            h2, zrows, zs)

    b2p = jnp.pad(b2, (0, DH - D_OUT))
    out_full = pl.pallas_call(
        _comb2_body,
        out_shape=jax.ShapeDtypeStruct((NP, DH), f32),
        compiler_params=pltpu.CompilerParams(vmem_limit_bytes=110 * 2**20),
    )(outp2, sp2.reshape(2, NP), a2s, a2d, h2, b2p.reshape(1, DH))

    return out_full[:N, :D_OUT]
